# Initial kernel scaffold; baseline (speedup 1.0000x reference)
import jax
import jax.numpy as jnp
from jax.experimental import pallas as pl


def kernel(x, edge_index, edge_attr, attention, W_cheb0, W_cheb1, b_cheb, Wz, bz, Wr, br, Wh, bh, Lz_w, Lz_b, Lr_w, Lr_b, Lh_w, Lh_b, W1, b1, W2, b2):
    raise NotImplementedError("write your pallas kernel here")


# speedup vs baseline: 24.1915x; 24.1915x over previous
import functools

import jax
import jax.numpy as jnp
from jax import lax
from jax.experimental import pallas as pl
from jax.experimental.pallas import tpu as pltpu
from jax.experimental.pallas import tpu_sc as plsc

N = 10000
E = 320000
F_IN = 128
F_OUT = 256
PERIODS = 8
HID = 128
OUT_DIM = 32

NC = 2
NS = 16
NPAD = 10240
EPAD = 327680
CH = 128
NBUF = 4
CHUNKS = EPAD // NS // CH
DEG_CH = 2048
DEG_CHUNKS = EPAD // NS // DEG_CH
NORM_CH = 2048
NORM_CHUNKS = EPAD // (NC * NS) // NORM_CH
NPT = NPAD // NS

_mesh = functools.partial(
    plsc.VectorSubcoreMesh, core_axis_name="c", subcore_axis_name="s",
    num_cores=NC, num_subcores=NS)

_f32 = jnp.float32
_i32 = jnp.int32


def _nrsqrt(v):
  i = lax.bitcast_convert_type(v, _i32)
  i = jnp.int32(0x5F3759DF) - lax.shift_right_arithmetic(i, 1)
  y = lax.bitcast_convert_type(i, _f32)
  for _ in range(4):
    y = y * (1.5 - 0.5 * v * y * y)
  return y


def _prep_body(row_h, col_h, w_h, nsl_h, nc_h, d2_h,
               parts_sh, dsl_sh, dc_sh,
               colv, wv, rowv, degloc, dslloc, dcloc,
               tmpa, tmpb, tmpc, nslv, ncv):
  c = lax.axis_index("c")
  s = lax.axis_index("s")

  def _zb(j, _):
    degloc[pl.ds(j * 16, 16)] = jnp.zeros((16,), _f32)
    return 0
  lax.fori_loop(0, NPAD // 16, _zb, 0)

  def _chb(kk, _):
    base = s * (EPAD // NS) + kk * DEG_CH
    pltpu.sync_copy(col_h.at[pl.ds(base, DEG_CH)], colv)
    pltpu.sync_copy(w_h.at[pl.ds(base, DEG_CH)], wv)
    def _eb(i, _):
      sl = pl.ds(i * 16, 16)
      plsc.addupdate_scatter(degloc, [colv[sl]], wv[sl])
      return 0
    lax.fori_loop(0, DEG_CH // 16, _eb, 0)
    return 0
  lax.fori_loop(0, DEG_CHUNKS, _chb, 0)

  pltpu.sync_copy(degloc, parts_sh.at[pl.ds(s * NPAD, NPAD)])
  plsc.subcore_barrier()
  def _zt(j, _):
    tmpa[pl.ds(j * 16, 16)] = jnp.zeros((16,), _f32)
    return 0
  lax.fori_loop(0, NPT // 16, _zt, 0)
  def _rb(t, _):
    pltpu.sync_copy(parts_sh.at[pl.ds(t * NPAD + s * NPT, NPT)], tmpb)
    for k in range(NPT // 16):
      sl = pl.ds(k * 16, 16)
      tmpa[sl] = tmpa[sl] + tmpb[sl]
    return 0
  lax.fori_loop(0, NS, _rb, 0)

  for k in range(NPT // 16):
    sl = pl.ds(k * 16, 16)
    dv = tmpa[sl]
    ysl = _nrsqrt(dv + 1.0)
    yc = jnp.where(dv > 0, _nrsqrt(jnp.maximum(dv, 1e-30)), 0.0)
    tmpb[sl] = ysl
    tmpc[sl] = ysl * ysl
    tmpa[sl] = yc
  pltpu.sync_copy(tmpb, dsl_sh.at[pl.ds(s * NPT, NPT)])
  pltpu.sync_copy(tmpa, dc_sh.at[pl.ds(s * NPT, NPT)])
  @pl.when(c == 0)
  def _():
    pltpu.sync_copy(tmpc, d2_h.at[pl.ds(s * NPT, NPT)])
  plsc.subcore_barrier()

  pltpu.sync_copy(dsl_sh, dslloc)
  pltpu.sync_copy(dc_sh, dcloc)
  wid = s * NC + c
  def _chd(kk, _):
    base = wid * (EPAD // (NC * NS)) + kk * NORM_CH
    pltpu.sync_copy(row_h.at[pl.ds(base, NORM_CH)], rowv)
    pltpu.sync_copy(col_h.at[pl.ds(base, NORM_CH)], colv)
    pltpu.sync_copy(w_h.at[pl.ds(base, NORM_CH)], wv)
    def _nb(i, _):
      sl = pl.ds(i * 16, 16)
      rv = rowv[sl]
      cv = colv[sl]
      wvv = wv[sl]
      g1 = plsc.load_gather(dslloc, [rv])
      g2 = plsc.load_gather(dslloc, [cv])
      nslv[sl] = g1 * wvv * g2
      g3 = plsc.load_gather(dcloc, [rv])
      g4 = plsc.load_gather(dcloc, [cv])
      ncv[sl] = g3 * wvv * g4
      return 0
    lax.fori_loop(0, NORM_CH // 16, _nb, 0)
    pltpu.sync_copy(nslv, nsl_h.at[pl.ds(base, NORM_CH)])
    pltpu.sync_copy(ncv, nc_h.at[pl.ds(base, NORM_CH)])
    return 0
  lax.fori_loop(0, NORM_CHUNKS, _chd, 0)


_prep_call = pl.kernel(
    _prep_body,
    out_type=[jax.ShapeDtypeStruct((EPAD,), _f32),
              jax.ShapeDtypeStruct((EPAD,), _f32),
              jax.ShapeDtypeStruct((NPAD,), _f32)],
    mesh=_mesh(),
    compiler_params=pltpu.CompilerParams(needs_layout_passes=False),
    scratch_types=[
        pltpu.VMEM_SHARED((NS * NPAD,), _f32),
        pltpu.VMEM_SHARED((NPAD,), _f32),
        pltpu.VMEM_SHARED((NPAD,), _f32),
        pltpu.VMEM((DEG_CH,), _i32),
        pltpu.VMEM((DEG_CH,), _f32),
        pltpu.VMEM((NORM_CH,), _i32),
        pltpu.VMEM((NPAD,), _f32),
        pltpu.VMEM((NPAD,), _f32),
        pltpu.VMEM((NPAD,), _f32),
        pltpu.VMEM((NPT,), _f32),
        pltpu.VMEM((NPT,), _f32),
        pltpu.VMEM((NPT,), _f32),
        pltpu.VMEM((NORM_CH,), _f32),
        pltpu.VMEM((NORM_CH,), _f32),
    ],
)


def _prop_body(xtab_h, rc2_h, nsl_h, nc_h, zeros_h, s_h,
               acc_sh, rcv2,
               idxb, colb, nslb, ncb, rows_b, msg_b, sem_g, sem_s):
  c = lax.axis_index("c")
  s = lax.axis_index("s")

  pltpu.sync_copy(rc2_h.at[pl.ds(s * CHUNKS, CHUNKS)], rcv2)
  ebase = s * (EPAD // NS)

  def _prep_gather(k, b, base_off):
    for i in range(CH // 16):
      sl = pl.ds(i * 16, 16)
      v = rcv2[k, sl]
      idxb[b][sl] = lax.bitwise_and(v, 16383) + base_off
    pltpu.async_copy(nsl_h.at[pl.ds(ebase + k * CH, CH)], nslb[b], sem_g[b])
    pltpu.async_copy(nc_h.at[pl.ds(ebase + k * CH, CH)], ncb[b], sem_g[b])
    pltpu.async_copy(xtab_h.at[idxb[b]], rows_b[b], sem_g[b])

  def _wait_gather(k, b):
    pltpu.make_async_copy(nsl_h.at[pl.ds(ebase, CH)], nslb[b], sem_g[b]).wait()
    pltpu.make_async_copy(nc_h.at[pl.ds(ebase, CH)], ncb[b], sem_g[b]).wait()
    pltpu.make_async_copy(xtab_h.at[idxb[b]], rows_b[b], sem_g[b]).wait()

  def _pass(hh):
    q = hh * NC + c
    pltpu.sync_copy(zeros_h, acc_sh.at[pl.ds(s * NPT, NPT)])
    plsc.subcore_barrier()
    base_off = q * NPAD

    for b in range(NBUF - 1):
      _prep_gather(b, b, base_off)

    def _chunk_quad(kk, _):
      for b in range(NBUF):
        k = kk * NBUF + b
        nb = (b + NBUF - 1) % NBUF
        @pl.when(k + NBUF - 1 < CHUNKS)
        def _():
          _prep_gather(k + NBUF - 1, nb, base_off)
        _wait_gather(k, b)
        @pl.when(k >= NBUF)
        def _():
          pltpu.make_async_copy(msg_b[b], acc_sh.at[colb[b].at[0]],
                                sem_s[b]).wait()
        for i in range(CH // 16):
          sl = pl.ds(i * 16, 16)
          colb[b][0, sl] = lax.shift_right_logical(rcv2[k, sl], 14)
        @plsc.parallel_loop(0, CH, unroll=16)
        def _edge(e):
          eb = jnp.broadcast_to(e, (16,))
          nsl = plsc.load_gather(nslb[b], [eb])
          ncc = plsc.load_gather(ncb[b], [eb])
          rr = plsc.unpack(rows_b[b][e], format=plsc.PackFormat.INTERLEAVED)
          for j in range(2):
            sl = pl.ds(j * 16, 16)
            msg_b[b][e, 0, sl] = nsl * rr[j]
            msg_b[b][e, 1, sl] = ncc * rr[j]
        pltpu.async_copy(msg_b[b], acc_sh.at[colb[b].at[0]], sem_s[b],
                         add=True)
      return 0

    lax.fori_loop(0, CHUNKS // NBUF, _chunk_quad, 0)
    for b in range(NBUF):
      pltpu.make_async_copy(msg_b[b], acc_sh.at[colb[b].at[0]],
                            sem_s[b]).wait()
    plsc.subcore_barrier()
    pltpu.sync_copy(acc_sh.at[pl.ds(s * NPT, NPT)],
                    s_h.at[q, pl.ds(s * NPT, NPT)])
    return 0

  def _half(hh, _):
    _pass(hh)
    return 0
  lax.fori_loop(0, 2, _half, 0)


_prop_call = pl.kernel(
    _prop_body,
    out_type=[jax.ShapeDtypeStruct((4, NPAD, 2, 32), _f32)],
    mesh=_mesh(),
    compiler_params=pltpu.CompilerParams(needs_layout_passes=False,
                                         use_tc_tiling_on_sc=False),
    scratch_types=[
        pltpu.VMEM_SHARED((NPAD, 2, 32), _f32),
        pltpu.VMEM((CHUNKS, CH), _i32),
        [pltpu.VMEM((CH,), _i32)] * NBUF,
        [pltpu.VMEM((1, CH), _i32)] * NBUF,
        [pltpu.VMEM((CH,), _f32)] * NBUF,
        [pltpu.VMEM((CH,), _f32)] * NBUF,
        [pltpu.VMEM((CH, 32), jnp.bfloat16)] * NBUF,
        [pltpu.VMEM((CH, 2, 32), _f32)] * NBUF,
        [pltpu.SemaphoreType.DMA] * NBUF,
        [pltpu.SemaphoreType.DMA] * NBUF,
    ],
)


NB = NPAD // 512


def _bdot(a, b):
  return jnp.dot(a.astype(jnp.bfloat16), b.astype(jnp.bfloat16),
                 preferred_element_type=_f32)

def _dense_body(acc_in, x_ref, s_ref, d2_ref, att_ref, pidx_ref,
                w0_ref, w1c_ref, bch_ref,
                wz_ref, wr_ref, wh_ref, lz_ref, lr_ref, lh_ref,
                bz_ref, br_ref, bh_ref, lzb_ref, lrb_ref, lhb_ref,
                acc_out,
                wzp, wrp, whp, bzp, brp, bhp):
  b = pl.program_id(0)

  @pl.when(b == 0)
  def _():
    lz1 = lz_ref[:F_OUT]
    lr1 = lr_ref[:F_OUT]
    lh1 = lh_ref[:F_OUT]
    wzp[...] = jnp.dot(wz_ref[...], lz1, preferred_element_type=_f32)
    wrp[...] = jnp.dot(wr_ref[...], lr1, preferred_element_type=_f32)
    whp[...] = jnp.dot(wh_ref[...], lh1, preferred_element_type=_f32)
    bzp[...] = jnp.dot(bz_ref[...], lz1, preferred_element_type=_f32) + lzb_ref[...]
    brp[...] = jnp.dot(br_ref[...], lr1, preferred_element_type=_f32) + lrb_ref[...]
    bhp[...] = jnp.dot(bh_ref[...], lh1, preferred_element_type=_f32) + lhb_ref[...]

  xp = x_ref[...]
  sblk = s_ref[...]
  se = jnp.concatenate([sblk[q, :, 0, :] for q in range(4)], axis=-1)
  sc = jnp.concatenate([sblk[q, :, 1, :] for q in range(4)], axis=-1)
  d2b = d2_ref[...]
  s_sl = se + d2b * xp

  h = _bdot(xp, w0_ref[...]) - _bdot(sc, w1c_ref[...]) + bch_ref[...]
  lz2 = lz_ref[F_OUT:]
  lr2 = lr_ref[F_OUT:]
  lh2 = lh_ref[F_OUT:]
  z = jax.nn.sigmoid(_bdot(s_sl, wzp[...]) + _bdot(h, lz2) + bzp[...])
  r = jax.nn.sigmoid(_bdot(s_sl, wrp[...]) + _bdot(h, lr2) + brp[...])
  ht = jnp.tanh(_bdot(s_sl, whp[...]) + _bdot(h * r, lh2) + bhp[...])
  hn = z * h + (1.0 - z) * ht

  a = att_ref[...]
  ea = jnp.exp(a - jnp.max(a))
  probs = ea / jnp.sum(ea)
  pr = jnp.sum(jnp.where(
      lax.broadcasted_iota(_i32, (1, PERIODS), 1) == pidx_ref[0], probs, 0.0))
  acc_out[...] = acc_in[...] + pr * hn


def _full(shape):
  return pl.BlockSpec(shape, lambda b: (0,) * len(shape))


_dense_call = pl.pallas_call(
    _dense_body,
    grid=(NB,),
    in_specs=[
        pl.BlockSpec((512, F_OUT), lambda b: (b, 0)),
        pl.BlockSpec((512, 128), lambda b: (b, 0)),
        pl.BlockSpec((4, 512, 2, 32), lambda b: (0, b, 0, 0)),
        pl.BlockSpec((512, 1), lambda b: (b, 0)),
        _full((1, PERIODS)),
        pl.BlockSpec(memory_space=pltpu.SMEM),
        _full((F_IN, F_OUT)), _full((F_IN, F_OUT)), _full((1, F_OUT)),
        _full((F_IN, F_OUT)), _full((F_IN, F_OUT)), _full((F_IN, F_OUT)),
        _full((2 * F_OUT, F_OUT)), _full((2 * F_OUT, F_OUT)), _full((2 * F_OUT, F_OUT)),
        _full((1, F_OUT)), _full((1, F_OUT)), _full((1, F_OUT)),
        _full((1, F_OUT)), _full((1, F_OUT)), _full((1, F_OUT)),
    ],
    out_specs=[pl.BlockSpec((512, F_OUT), lambda b: (b, 0))],
    out_shape=[jax.ShapeDtypeStruct((NPAD, F_OUT), _f32)],
    input_output_aliases={0: 0},
    scratch_shapes=[
        pltpu.VMEM((F_IN, F_OUT), _f32), pltpu.VMEM((F_IN, F_OUT), _f32),
        pltpu.VMEM((F_IN, F_OUT), _f32),
        pltpu.VMEM((1, F_OUT), _f32), pltpu.VMEM((1, F_OUT), _f32),
        pltpu.VMEM((1, F_OUT), _f32),
    ],
)


def _mlp_body(acc_ref, w1_ref, b1_ref, w2_ref, b2_ref, out_ref):
  hrelu = jnp.maximum(acc_ref[...], 0.0)
  h1 = jnp.maximum(_bdot(hrelu, w1_ref[...]) + b1_ref[...], 0.0)
  out_ref[...] = _bdot(h1, w2_ref[...]) + b2_ref[...]


_mlp_call = pl.pallas_call(
    _mlp_body,
    grid=(NB,),
    in_specs=[
        pl.BlockSpec((512, F_OUT), lambda b: (b, 0)),
        _full((F_OUT, HID)), _full((1, HID)),
        _full((HID, OUT_DIM)), _full((1, OUT_DIM)),
    ],
    out_specs=[pl.BlockSpec((512, OUT_DIM), lambda b: (b, 0))],
    out_shape=[jax.ShapeDtypeStruct((NPAD, OUT_DIM), _f32)],
)


def kernel(x, edge_index, edge_attr, attention, W_cheb0, W_cheb1, b_cheb,
           Wz, bz, Wr, br, Wh, bh, Lz_w, Lz_b, Lr_w, Lr_b, Lh_w, Lh_b,
           W1, b1, W2, b2):
  row = edge_index[0].astype(_i32)
  col = edge_index[1].astype(_i32)
  pad = EPAD - E
  row_p = jnp.pad(row, (0, pad))
  col_p = jnp.pad(col, (0, pad))
  w_p = jnp.pad(edge_attr.astype(_f32), (0, pad))

  xt = jnp.transpose(x, (2, 0, 1))
  xpad = jnp.pad(xt, ((0, 0), (0, NPAD - N), (0, 0)))
  xtab4 = (xpad.reshape(PERIODS, NPAD, 4, 32)
           .transpose(0, 2, 1, 3)
           .reshape(PERIODS, 4 * NPAD, 2, 16)
           .transpose(0, 1, 3, 2)
           .reshape(PERIODS, 4 * NPAD, 32)
           .astype(jnp.bfloat16))
  zeros = jnp.zeros((NPT, 2, 32), _f32)

  nsl, ncn, d2 = _prep_call(row_p, col_p, w_p)
  d2 = d2.reshape(NPAD, 1)
  rc2 = (row_p | (col_p << 14)).reshape(EPAD // CH, CH)
  att = attention.reshape(1, PERIODS)
  bchr = b_cheb.reshape(1, F_OUT)
  bzr, brr, bhr = (bz.reshape(1, F_OUT), br.reshape(1, F_OUT),
                   bh.reshape(1, F_OUT))
  lzbr, lrbr, lhbr = (Lz_b.reshape(1, F_OUT), Lr_b.reshape(1, F_OUT),
                      Lh_b.reshape(1, F_OUT))
  s_outs = [_prop_call(xtab4[p], rc2, nsl, ncn, zeros)[0]
            for p in range(PERIODS)]
  acc = jnp.zeros((NPAD, F_OUT), _f32)
  for p in range(PERIODS):
    (acc,) = _dense_call(
        acc, xpad[p], s_outs[p], d2, att, jnp.array([p], _i32),
        W_cheb0, W_cheb1, bchr,
        Wz, Wr, Wh, Lz_w, Lr_w, Lh_w,
        bzr, brr, bhr, lzbr, lrbr, lhbr)
  (out_p,) = _mlp_call(acc, W1, b1.reshape(1, HID), W2,
                       b2.reshape(1, OUT_DIM))
  return out_p[:N], acc[:N]
